# baseline (speedup 1.0000x reference)
import functools

import jax
import jax.numpy as jnp
from jax import lax
from jax.experimental import pallas as pl
from jax.experimental.pallas import tpu as pltpu
from jax.experimental.pallas import tpu_sc as plsc

L = 16


def _pack_words(t):
    n, m = t.shape
    t4 = t.reshape(n, m // 32, 2, L)
    a = lax.bitcast_convert_type(
        t4[:, :, 0, :].astype(jnp.bfloat16), jnp.uint16).astype(jnp.uint32)
    b = lax.bitcast_convert_type(
        t4[:, :, 1, :].astype(jnp.bfloat16), jnp.uint16).astype(jnp.uint32)
    w = a | (b << 16)
    return lax.bitcast_convert_type(w, jnp.int32).reshape(n, m // 2)


def _tables_body(x_ref, w_ref, b_ref, ysb_ref, y_ref):
    y = lax.dot_general(
        x_ref[...], w_ref[...], (((1,), (0,)), ((), ())),
        precision=lax.Precision.HIGHEST,
        preferred_element_type=jnp.float32,
    )
    ysb_ref[...] = y + b_ref[...]
    y_ref[...] = y


def _build_tables(x, W1, b1_row):
    n, d = x.shape
    h = W1.shape[1]
    ysb, y = pl.pallas_call(
        _tables_body,
        out_shape=(
            jax.ShapeDtypeStruct((n, h), jnp.float32),
            jax.ShapeDtypeStruct((n, h), jnp.float32),
        ),
    )(x, W1, b1_row)
    ts = _pack_words(ysb)
    td = jnp.concatenate([_pack_words(y), _pack_words(x)], axis=1)
    return ts, td


def _make_edge_kernel(e_pad, n_workers, chunk, d, h):
    epw = e_pad // n_workers
    n_chunks = epw // chunk
    nbuf = 6
    n_quads = n_chunks // nbuf
    n_tail = n_chunks % nbuf
    nbh = h // 32
    nbd = d // 32
    hw = h // 2
    dw = d // 2

    mesh = plsc.VectorSubcoreMesh(core_axis_name="c", subcore_axis_name="s")

    @functools.partial(
        pl.kernel,
        out_type=jax.ShapeDtypeStruct((e_pad, d), jnp.float32),
        mesh=mesh,
        scratch_types=[
            pltpu.VMEM((epw,), jnp.int32),
            pltpu.VMEM((epw,), jnp.int32),
            pltpu.VMEM((nbuf, chunk, hw), jnp.int32),
            pltpu.VMEM((nbuf, chunk, hw + dw), jnp.int32),
            pltpu.VMEM((nbuf, chunk, d), jnp.float32),
            pltpu.VMEM((hw,), jnp.int32),
            pltpu.VMEM((L,), jnp.float32),
            [pltpu.SemaphoreType.DMA] * nbuf,
            [pltpu.SemaphoreType.DMA] * nbuf,
        ],
        compiler_params=pltpu.CompilerParams(
            needs_layout_passes=False, use_tc_tiling_on_sc=False),
    )
    def edge_kernel(ysb_hbm, xy_hbm, src_hbm, dst_hbm, w2_hbm, b2_hbm,
                    out_hbm, src_v, dst_v, ys_v, xy_v, out_v, w2_v,
                    b2_v, gsem, osem):
        n_cores = 2
        wid = lax.axis_index("s") * n_cores + lax.axis_index("c")
        base = wid * epw
        pltpu.sync_copy(src_hbm.at[pl.ds(base, epw)], src_v)
        pltpu.sync_copy(dst_hbm.at[pl.ds(base, epw)], dst_v)
        pltpu.sync_copy(w2_hbm, w2_v)
        pltpu.sync_copy(b2_hbm, b2_v)
        b2 = b2_v[...]
        w2b = [plsc.bitcast(w2_v[pl.ds(j * L, L)], jnp.bfloat16)
               for j in range(nbh)]

        def issue_gathers(slot, ci):
            off = ci * chunk
            pltpu.async_copy(ysb_hbm.at[src_v.at[pl.ds(off, chunk)]],
                             ys_v.at[slot], gsem[slot])
            pltpu.async_copy(xy_hbm.at[dst_v.at[pl.ds(off, chunk)]],
                             xy_v.at[slot], gsem[slot])

        def wait_gathers(slot):
            pltpu.make_async_copy(ysb_hbm.at[pl.ds(0, chunk)],
                                  ys_v.at[slot], gsem[slot]).wait()
            pltpu.make_async_copy(xy_hbm.at[pl.ds(0, chunk)],
                                  xy_v.at[slot], gsem[slot]).wait()

        def issue_store(slot, ci):
            pltpu.async_copy(out_v.at[slot],
                             out_hbm.at[pl.ds(base + ci * chunk, chunk)],
                             osem[slot])

        def wait_store(slot):
            pltpu.make_async_copy(out_v.at[slot],
                                  out_hbm.at[pl.ds(base, chunk)],
                                  osem[slot]).wait()

        def compute(slot):
            bzero = jnp.asarray(0, jnp.bfloat16)

            @plsc.parallel_loop(0, chunk, unroll=2)
            def _(e):
                acc32 = None
                for j in range(nbh):
                    ys32 = plsc.bitcast(ys_v[slot, e, pl.ds(L * j, L)],
                                        jnp.bfloat16)
                    yd32 = plsc.bitcast(xy_v[slot, e, pl.ds(L * j, L)],
                                        jnp.bfloat16)
                    p = jnp.maximum(ys32 - yd32, bzero) * w2b[j]
                    acc32 = p if acc32 is None else acc32 + p
                aa, ab = plsc.unpack(acc32,
                                     format=plsc.PackFormat.INTERLEAVED)
                s = jnp.sum(aa + ab)
                att = jnp.exp(-jnp.maximum(s + b2, 0.0))
                for j in range(nbd):
                    xa, xb = plsc.unpack(
                        plsc.bitcast(xy_v[slot, e, pl.ds(hw + L * j, L)],
                                     jnp.bfloat16),
                        format=plsc.PackFormat.INTERLEAVED)
                    out_v[slot, e, pl.ds(32 * j, L)] = xa * att
                    out_v[slot, e, pl.ds(32 * j + L, L)] = xb * att

        for s in range(min(nbuf, n_chunks)):
            issue_gathers(s, s)

        def quad_body(q, _):
            c = nbuf * q
            for s in range(nbuf):
                wait_gathers(s)

                @pl.when(q > 0)
                def _():
                    wait_store(s)

                compute(s)
                issue_store(s, c + s)
                cn = c + nbuf + s

                @pl.when(cn < n_chunks)
                def _():
                    issue_gathers(s, cn)
            return 0

        lax.fori_loop(0, n_quads, quad_body, 0)
        for s in range(n_tail):
            wait_gathers(s)
            if n_quads > 0:
                wait_store(s)
            compute(s)
            issue_store(s, nbuf * n_quads + s)
        for s in range(min(nbuf, n_chunks)):
            wait_store(s)

    return edge_kernel


def kernel(x, edge_index, W1, b1, W2, b2):
    n, d = x.shape
    h = W1.shape[1]
    e = edge_index.shape[1]

    src = edge_index[0].astype(jnp.int32)
    dst = edge_index[1].astype(jnp.int32)

    n_workers = 32
    chunk = 40
    step = n_workers * chunk
    e_pad = ((e + step - 1) // step) * step
    if e_pad != e:
        src = jnp.pad(src, (0, e_pad - e))
        dst = jnp.pad(dst, (0, e_pad - e))

    ysb, xy = _build_tables(x, W1, b1.reshape(1, h))
    w2_pk = _pack_words(W2[:, 0].reshape(1, h)).reshape(h // 2)
    b2v = jnp.broadcast_to(b2, (L,))

    out = _make_edge_kernel(e_pad, n_workers, chunk, d, h)(
        ysb, xy, src, dst, w2_pk, b2v)
    return out[:e] if e_pad != e else out


# speedup vs baseline: 1.0128x; 1.0128x over previous
import functools

import jax
import jax.numpy as jnp
from jax import lax
from jax.experimental import pallas as pl
from jax.experimental.pallas import tpu as pltpu
from jax.experimental.pallas import tpu_sc as plsc

L = 16


def _pack_words(t):
    n, m = t.shape
    t4 = t.reshape(n, m // 32, 2, L)
    a = lax.bitcast_convert_type(
        t4[:, :, 0, :].astype(jnp.bfloat16), jnp.uint16).astype(jnp.uint32)
    b = lax.bitcast_convert_type(
        t4[:, :, 1, :].astype(jnp.bfloat16), jnp.uint16).astype(jnp.uint32)
    w = a | (b << 16)
    return lax.bitcast_convert_type(w, jnp.int32).reshape(n, m // 2)


def _tables_body(x_ref, w_ref, b_ref, ysb_ref, y_ref):
    y = lax.dot_general(
        x_ref[...], w_ref[...], (((1,), (0,)), ((), ())),
        precision=lax.Precision.HIGHEST,
        preferred_element_type=jnp.float32,
    )
    ysb_ref[...] = y + b_ref[...]
    y_ref[...] = y


def _build_tables(x, W1, b1_row):
    n, d = x.shape
    h = W1.shape[1]
    ysb, y = pl.pallas_call(
        _tables_body,
        out_shape=(
            jax.ShapeDtypeStruct((n, h), jnp.float32),
            jax.ShapeDtypeStruct((n, h), jnp.float32),
        ),
    )(x, W1, b1_row)
    ts = _pack_words(ysb)
    td = jnp.concatenate([_pack_words(y), _pack_words(x)], axis=1)
    return ts, td


def _make_edge_kernel(e_pad, n_workers, chunk, d, h):
    epw = e_pad // n_workers
    n_chunks = epw // chunk
    n_quads = n_chunks // 4
    n_tail = n_chunks % 4
    nbuf = 4
    nbh = h // 32
    nbd = d // 32
    hw = h // 2
    dw = d // 2

    mesh = plsc.VectorSubcoreMesh(core_axis_name="c", subcore_axis_name="s")

    @functools.partial(
        pl.kernel,
        out_type=jax.ShapeDtypeStruct((e_pad, d), jnp.float32),
        mesh=mesh,
        scratch_types=[
            pltpu.VMEM((epw,), jnp.int32),
            pltpu.VMEM((epw,), jnp.int32),
            pltpu.VMEM((nbuf, chunk, hw), jnp.int32),
            pltpu.VMEM((nbuf, chunk, hw + dw), jnp.int32),
            pltpu.VMEM((nbuf, chunk, d), jnp.float32),
            pltpu.VMEM((hw,), jnp.int32),
            pltpu.VMEM((L,), jnp.float32),
            [pltpu.SemaphoreType.DMA] * nbuf,
            [pltpu.SemaphoreType.DMA] * nbuf,
        ],
        compiler_params=pltpu.CompilerParams(
            needs_layout_passes=False, use_tc_tiling_on_sc=False),
    )
    def edge_kernel(ysb_hbm, xy_hbm, src_hbm, dst_hbm, w2_hbm, b2_hbm,
                    out_hbm, src_v, dst_v, ys_v, xy_v, out_v, w2_v,
                    b2_v, gsem, osem):
        n_cores = 2
        wid = lax.axis_index("s") * n_cores + lax.axis_index("c")
        base = wid * epw
        pltpu.sync_copy(src_hbm.at[pl.ds(base, epw)], src_v)
        pltpu.sync_copy(dst_hbm.at[pl.ds(base, epw)], dst_v)
        pltpu.sync_copy(w2_hbm, w2_v)
        pltpu.sync_copy(b2_hbm, b2_v)
        b2 = b2_v[...]
        w2b = [plsc.bitcast(w2_v[pl.ds(j * L, L)], jnp.bfloat16)
               for j in range(nbh)]

        ha = 16
        hb = chunk - ha

        def issue_gathers(slot, ci):
            off = ci * chunk
            pltpu.async_copy(ysb_hbm.at[src_v.at[pl.ds(off, ha)]],
                             ys_v.at[slot, pl.ds(0, ha)], gsem[slot])
            pltpu.async_copy(ysb_hbm.at[src_v.at[pl.ds(off + ha, hb)]],
                             ys_v.at[slot, pl.ds(ha, hb)], gsem[slot])
            pltpu.async_copy(xy_hbm.at[dst_v.at[pl.ds(off, ha)]],
                             xy_v.at[slot, pl.ds(0, ha)], gsem[slot])
            pltpu.async_copy(xy_hbm.at[dst_v.at[pl.ds(off + ha, hb)]],
                             xy_v.at[slot, pl.ds(ha, hb)], gsem[slot])

        def wait_gathers(slot):
            pltpu.make_async_copy(ysb_hbm.at[pl.ds(0, chunk)],
                                  ys_v.at[slot], gsem[slot]).wait()
            pltpu.make_async_copy(xy_hbm.at[pl.ds(0, chunk)],
                                  xy_v.at[slot], gsem[slot]).wait()

        def issue_store(slot, ci):
            pltpu.async_copy(out_v.at[slot],
                             out_hbm.at[pl.ds(base + ci * chunk, chunk)],
                             osem[slot])

        def wait_store(slot):
            pltpu.make_async_copy(out_v.at[slot],
                                  out_hbm.at[pl.ds(base, chunk)],
                                  osem[slot]).wait()

        def compute(slot):
            bzero = jnp.asarray(0, jnp.bfloat16)

            @plsc.parallel_loop(0, chunk, unroll=2)
            def _(e):
                acc32 = None
                for j in range(nbh):
                    ys32 = plsc.bitcast(ys_v[slot, e, pl.ds(L * j, L)],
                                        jnp.bfloat16)
                    yd32 = plsc.bitcast(xy_v[slot, e, pl.ds(L * j, L)],
                                        jnp.bfloat16)
                    p = jnp.maximum(ys32 - yd32, bzero) * w2b[j]
                    acc32 = p if acc32 is None else acc32 + p
                aa, ab = plsc.unpack(acc32,
                                     format=plsc.PackFormat.INTERLEAVED)
                s = jnp.sum(aa + ab)
                att = jnp.exp(-jnp.maximum(s + b2, 0.0))
                for j in range(nbd):
                    xa, xb = plsc.unpack(
                        plsc.bitcast(xy_v[slot, e, pl.ds(hw + L * j, L)],
                                     jnp.bfloat16),
                        format=plsc.PackFormat.INTERLEAVED)
                    out_v[slot, e, pl.ds(32 * j, L)] = xa * att
                    out_v[slot, e, pl.ds(32 * j + L, L)] = xb * att

        for s in range(min(nbuf, n_chunks)):
            issue_gathers(s, s)

        def quad_body(q, _):
            c = nbuf * q
            for s in range(nbuf):
                wait_gathers(s)

                @pl.when(q > 0)
                def _():
                    wait_store(s)

                compute(s)
                issue_store(s, c + s)
                cn = c + nbuf + s

                @pl.when(cn < n_chunks)
                def _():
                    issue_gathers(s, cn)
            return 0

        lax.fori_loop(0, n_quads, quad_body, 0)
        for s in range(n_tail):
            wait_gathers(s)
            if n_quads > 0:
                wait_store(s)
            compute(s)
            issue_store(s, nbuf * n_quads + s)
        for s in range(min(nbuf, n_chunks)):
            wait_store(s)

    return edge_kernel


def kernel(x, edge_index, W1, b1, W2, b2):
    n, d = x.shape
    h = W1.shape[1]
    e = edge_index.shape[1]

    src = edge_index[0].astype(jnp.int32)
    dst = edge_index[1].astype(jnp.int32)

    n_workers = 32
    chunk = 40
    step = n_workers * chunk
    e_pad = ((e + step - 1) // step) * step
    if e_pad != e:
        src = jnp.pad(src, (0, e_pad - e))
        dst = jnp.pad(dst, (0, e_pad - e))

    ysb, xy = _build_tables(x, W1, b1.reshape(1, h))
    w2_pk = _pack_words(W2[:, 0].reshape(1, h)).reshape(h // 2)
    b2v = jnp.broadcast_to(b2, (L,))

    out = _make_edge_kernel(e_pad, n_workers, chunk, d, h)(
        ysb, xy, src, dst, w2_pk, b2v)
    return out[:e] if e_pad != e else out


# speedup vs baseline: 1.0218x; 1.0088x over previous
import functools

import jax
import jax.numpy as jnp
from jax import lax
from jax.experimental import pallas as pl
from jax.experimental.pallas import tpu as pltpu
from jax.experimental.pallas import tpu_sc as plsc

L = 16


def _pack_words(t):
    n, m = t.shape
    t4 = t.reshape(n, m // 32, 2, L)
    a = lax.bitcast_convert_type(
        t4[:, :, 0, :].astype(jnp.bfloat16), jnp.uint16).astype(jnp.uint32)
    b = lax.bitcast_convert_type(
        t4[:, :, 1, :].astype(jnp.bfloat16), jnp.uint16).astype(jnp.uint32)
    w = a | (b << 16)
    return lax.bitcast_convert_type(w, jnp.int32).reshape(n, m // 2)


def _tables_body(x_ref, w_ref, b_ref, ysb_ref, y_ref):
    y = lax.dot_general(
        x_ref[...], w_ref[...], (((1,), (0,)), ((), ())),
        precision=lax.Precision.HIGHEST,
        preferred_element_type=jnp.float32,
    )
    ysb_ref[...] = y + b_ref[...]
    y_ref[...] = y


def _build_tables(x, W1, b1_row):
    n, d = x.shape
    h = W1.shape[1]
    ysb, y = pl.pallas_call(
        _tables_body,
        out_shape=(
            jax.ShapeDtypeStruct((n, h), jnp.float32),
            jax.ShapeDtypeStruct((n, h), jnp.float32),
        ),
    )(x, W1, b1_row)
    ts = _pack_words(ysb)
    td = jnp.concatenate([_pack_words(y), _pack_words(x)], axis=1)
    return ts, td


def _make_edge_kernel(e_pad, n_workers, chunk, d, h):
    epw = e_pad // n_workers
    n_chunks = epw // chunk
    nbuf = 3
    n_quads = n_chunks // nbuf
    n_tail = n_chunks % nbuf
    nbh = h // 32
    nbd = d // 32
    hw = h // 2
    dw = d // 2

    mesh = plsc.VectorSubcoreMesh(core_axis_name="c", subcore_axis_name="s")

    @functools.partial(
        pl.kernel,
        out_type=jax.ShapeDtypeStruct((e_pad, d), jnp.float32),
        mesh=mesh,
        scratch_types=[
            pltpu.VMEM((epw,), jnp.int32),
            pltpu.VMEM((epw,), jnp.int32),
            pltpu.VMEM((nbuf, chunk, hw), jnp.int32),
            pltpu.VMEM((nbuf, chunk, hw + dw), jnp.int32),
            pltpu.VMEM((nbuf, chunk, d), jnp.float32),
            pltpu.VMEM((hw,), jnp.int32),
            pltpu.VMEM((L,), jnp.float32),
            [pltpu.SemaphoreType.DMA] * nbuf,
            [pltpu.SemaphoreType.DMA] * nbuf,
        ],
        compiler_params=pltpu.CompilerParams(
            needs_layout_passes=False, use_tc_tiling_on_sc=False),
    )
    def edge_kernel(ysb_hbm, xy_hbm, src_hbm, dst_hbm, w2_hbm, b2_hbm,
                    out_hbm, src_v, dst_v, ys_v, xy_v, out_v, w2_v,
                    b2_v, gsem, osem):
        n_cores = 2
        wid = lax.axis_index("s") * n_cores + lax.axis_index("c")
        base = wid * epw
        pltpu.sync_copy(src_hbm.at[pl.ds(base, epw)], src_v)
        pltpu.sync_copy(dst_hbm.at[pl.ds(base, epw)], dst_v)
        pltpu.sync_copy(w2_hbm, w2_v)
        pltpu.sync_copy(b2_hbm, b2_v)
        b2 = b2_v[...]
        w2b = [plsc.bitcast(w2_v[pl.ds(j * L, L)], jnp.bfloat16)
               for j in range(nbh)]

        def issue_gathers(slot, ci):
            off = ci * chunk
            pltpu.async_copy(ysb_hbm.at[src_v.at[pl.ds(off, chunk)]],
                             ys_v.at[slot], gsem[slot])
            pltpu.async_copy(xy_hbm.at[dst_v.at[pl.ds(off, chunk)]],
                             xy_v.at[slot], gsem[slot])

        def wait_gathers(slot):
            pltpu.make_async_copy(ysb_hbm.at[pl.ds(0, chunk)],
                                  ys_v.at[slot], gsem[slot]).wait()
            pltpu.make_async_copy(xy_hbm.at[pl.ds(0, chunk)],
                                  xy_v.at[slot], gsem[slot]).wait()

        def issue_store(slot, ci):
            pltpu.async_copy(out_v.at[slot],
                             out_hbm.at[pl.ds(base + ci * chunk, chunk)],
                             osem[slot])

        def wait_store(slot):
            pltpu.make_async_copy(out_v.at[slot],
                                  out_hbm.at[pl.ds(base, chunk)],
                                  osem[slot]).wait()

        def compute(slot):
            bzero = jnp.asarray(0, jnp.bfloat16)

            @plsc.parallel_loop(0, chunk, unroll=2)
            def _(e):
                acc32 = None
                for j in range(nbh):
                    ys32 = plsc.bitcast(ys_v[slot, e, pl.ds(L * j, L)],
                                        jnp.bfloat16)
                    yd32 = plsc.bitcast(xy_v[slot, e, pl.ds(L * j, L)],
                                        jnp.bfloat16)
                    p = jnp.maximum(ys32 - yd32, bzero) * w2b[j]
                    acc32 = p if acc32 is None else acc32 + p
                aa, ab = plsc.unpack(acc32,
                                     format=plsc.PackFormat.INTERLEAVED)
                s = jnp.sum(aa + ab)
                att = jnp.exp(-jnp.maximum(s + b2, 0.0))
                for j in range(nbd):
                    xa, xb = plsc.unpack(
                        plsc.bitcast(xy_v[slot, e, pl.ds(hw + L * j, L)],
                                     jnp.bfloat16),
                        format=plsc.PackFormat.INTERLEAVED)
                    out_v[slot, e, pl.ds(32 * j, L)] = xa * att
                    out_v[slot, e, pl.ds(32 * j + L, L)] = xb * att

        for s in range(min(nbuf, n_chunks)):
            issue_gathers(s, s)

        def quad_body(q, _):
            c = nbuf * q
            for s in range(nbuf):
                wait_gathers(s)

                @pl.when(q > 0)
                def _():
                    wait_store(s)

                compute(s)
                issue_store(s, c + s)
                cn = c + nbuf + s

                @pl.when(cn < n_chunks)
                def _():
                    issue_gathers(s, cn)
            return 0

        lax.fori_loop(0, n_quads, quad_body, 0)
        for s in range(n_tail):
            wait_gathers(s)
            if n_quads > 0:
                wait_store(s)
            compute(s)
            issue_store(s, nbuf * n_quads + s)
        for s in range(min(nbuf, n_chunks)):
            wait_store(s)

    return edge_kernel


def kernel(x, edge_index, W1, b1, W2, b2):
    n, d = x.shape
    h = W1.shape[1]
    e = edge_index.shape[1]

    src = edge_index[0].astype(jnp.int32)
    dst = edge_index[1].astype(jnp.int32)

    n_workers = 32
    chunk = 80
    step = n_workers * chunk
    e_pad = ((e + step - 1) // step) * step
    if e_pad != e:
        src = jnp.pad(src, (0, e_pad - e))
        dst = jnp.pad(dst, (0, e_pad - e))

    ysb, xy = _build_tables(x, W1, b1.reshape(1, h))
    w2_pk = _pack_words(W2[:, 0].reshape(1, h)).reshape(h // 2)
    b2v = jnp.broadcast_to(b2, (L,))

    out = _make_edge_kernel(e_pad, n_workers, chunk, d, h)(
        ysb, xy, src, dst, w2_pk, b2v)
    return out[:e] if e_pad != e else out
